# Initial kernel scaffold; baseline (speedup 1.0000x reference)
import jax
import jax.numpy as jnp
from jax.experimental import pallas as pl


def kernel(x, edge_index, edge_sign, Wq, Wk, Wv, RF, RF_q):
    raise NotImplementedError("write your pallas kernel here")


# speedup vs baseline: 25.4707x; 25.4707x over previous
import functools

import jax
import jax.numpy as jnp
from jax import lax
from jax.experimental import pallas as pl
from jax.experimental.pallas import tpu as pltpu
from jax.experimental.pallas import tpu_sc as plsc

N = 10000
E = 320000
H = 8
D = 16
M = 8
IND = 128

NC = 2
NS = 16
NW = NC * NS
EPW = E // NW
NP = 10240
CH = 80
NCHUNK = EPW // CH
RPS = NP // NS

@functools.cache
def _mesh():
    return plsc.VectorSubcoreMesh(core_axis_name="c", subcore_axis_name="s",
                                  num_cores=NC, num_subcores=NS)


def _pre_body(x_ref, wv_ref, wq_ref, wk_ref, bdq_ref, bdk_ref,
              v_ref, qs_ref, phi_ref):
    xb = x_ref[...]
    v_ref[...] = jnp.dot(xb, wv_ref[...], preferred_element_type=jnp.float32)
    qb = jnp.dot(xb, wq_ref[...], preferred_element_type=jnp.float32)
    qr = jnp.dot(qb, bdq_ref[...], preferred_element_type=jnp.float32)
    cs = jnp.cos(qr) + jnp.sin(qr)
    r8 = lax.broadcasted_iota(jnp.int32, (H * M, 2 * H), 0) // M
    c8 = lax.broadcasted_iota(jnp.int32, (H * M, 2 * H), 1)
    sel = (r8 == c8).astype(jnp.float32)
    qs_ref[...] = jnp.dot(cs, sel, preferred_element_type=jnp.float32)
    kb = jnp.dot(xb, wk_ref[...], preferred_element_type=jnp.float32)
    kr = jnp.dot(kb, bdk_ref[...], preferred_element_type=jnp.float32)
    phi_ref[...] = jnp.cos(kr) + jnp.sin(kr)


def _k1(x, wv, wq, wk, bdq, bdk):
    blk = 2000
    g = N // blk
    return pl.pallas_call(
        _pre_body,
        grid=(g,),
        in_specs=[
            pl.BlockSpec((blk, IND), lambda i: (i, 0)),
            pl.BlockSpec((IND, IND), lambda i: (0, 0)),
            pl.BlockSpec((IND, IND), lambda i: (0, 0)),
            pl.BlockSpec((IND, IND), lambda i: (0, 0)),
            pl.BlockSpec((IND, H * M), lambda i: (0, 0)),
            pl.BlockSpec((IND, H * M), lambda i: (0, 0)),
        ],
        out_specs=[
            pl.BlockSpec((blk, IND), lambda i: (i, 0)),
            pl.BlockSpec((blk, 2 * H), lambda i: (i, 0)),
            pl.BlockSpec((blk, H * M), lambda i: (i, 0)),
        ],
        out_shape=[
            jax.ShapeDtypeStruct((N, IND), jnp.float32),
            jax.ShapeDtypeStruct((N, 2 * H), jnp.float32),
            jax.ShapeDtypeStruct((N, H * M), jnp.float32),
        ],
    )(x, wv, wq, wk, bdq, bdk)


def _stage_a_body(src_hbm, dst_hbm, es_hbm, phi_hbm, z32_hbm, tpart_hbm,
                  src_v, dst_v, sgn_v, phr_v, pay_v, tsh):
    cid = lax.axis_index("c")
    sid = lax.axis_index("s")
    wid = cid * NS + sid

    rows0 = sid * RPS
    pltpu.sync_copy(z32_hbm.at[pl.ds(rows0, RPS)], tsh.at[pl.ds(rows0, RPS)])

    iota = lax.iota(jnp.int32, 16)
    zero16 = jnp.zeros((16,), jnp.float32)
    for g in range(CH // 16):
        rows = iota + g * 16
        for cz in range(18, 32):
            plsc.store_scatter(pay_v, [rows, jnp.full((16,), cz, jnp.int32)],
                               zero16)
    plsc.subcore_barrier()

    def chunk(i, carry):
        off = wid * EPW + i * CH
        pltpu.sync_copy(src_hbm.at[pl.ds(off, CH)], src_v)
        pltpu.sync_copy(dst_hbm.at[pl.ds(off, CH)], dst_v)
        pltpu.sync_copy(es_hbm.at[pl.ds(off, CH)], sgn_v)
        pltpu.sync_copy(phi_hbm.at[src_v], phr_v)
        for g in range(CH // 16):
            rows = iota + g * 16
            sg = sgn_v[pl.ds(g * 16, 16)]
            pf = jnp.where(sg > 0, 1.0, 0.0)
            nf = 1.0 - pf
            for hh in range(H):
                ph = plsc.load_gather(phr_v,
                                      [rows, jnp.full((16,), hh, jnp.int32)])
                plsc.store_scatter(pay_v,
                                   [rows, jnp.full((16,), hh, jnp.int32)],
                                   ph * pf)
                plsc.store_scatter(pay_v,
                                   [rows, jnp.full((16,), H + hh, jnp.int32)],
                                   ph * nf)
            plsc.store_scatter(pay_v, [rows, jnp.full((16,), 16, jnp.int32)], pf)
            plsc.store_scatter(pay_v, [rows, jnp.full((16,), 17, jnp.int32)], nf)
        pltpu.sync_copy(pay_v, tsh.at[dst_v], add=True)
        return carry

    lax.fori_loop(0, NCHUNK, chunk, 0)
    plsc.subcore_barrier()
    pltpu.sync_copy(tsh.at[pl.ds(rows0, RPS)],
                    tpart_hbm.at[cid, pl.ds(rows0, RPS)])


def _k2(src, dst, edge_sign, phi_s, z32):
    f = pl.kernel(
        _stage_a_body,
        out_type=jax.ShapeDtypeStruct((NC, NP, 32), jnp.float32),
        mesh=_mesh(),
        compiler_params=pltpu.CompilerParams(needs_layout_passes=False, use_tc_tiling_on_sc=False),
        scratch_types=[
            pltpu.VMEM((CH,), jnp.int32),
            pltpu.VMEM((CH,), jnp.int32),
            pltpu.VMEM((CH,), jnp.int32),
            pltpu.VMEM((CH, H), jnp.float32),
            pltpu.VMEM((CH, 32), jnp.float32),
            pltpu.VMEM_SHARED((NP, 32), jnp.float32),
        ],
    )
    return f(src, dst, edge_sign, phi_s, z32)


def _t2_body(tp_ref, t2_ref):
    t = tp_ref[0] + tp_ref[1]
    cp = t[:, 0:H]
    cn = t[:, H:2 * H]
    dp = jnp.maximum(t[:, 16:17], 1.0)
    dn = jnp.maximum(t[:, 17:18], 1.0)
    t2_ref[...] = jnp.concatenate([cp / dp, cn / dn], axis=1)


def _k3(tpart):
    blk = 2000
    g = N // blk
    return pl.pallas_call(
        _t2_body,
        grid=(g,),
        in_specs=[pl.BlockSpec((NC, blk, 32), lambda i: (0, i, 0))],
        out_specs=pl.BlockSpec((blk, 2 * H), lambda i: (i, 0)),
        out_shape=jax.ShapeDtypeStruct((N, 2 * H), jnp.float32),
    )(tpart)


_LOG2E = 1.4426950408889634
_LN2_HI = 0.6931471824645996
_LN2_LO = -1.904654323148236e-09
_RND = 12582912.0


def _exp16(x):
    t = x * _LOG2E + _RND
    n = t - _RND
    r = x - n * _LN2_HI
    r = r - n * _LN2_LO
    p = jnp.float32(1.0 / 720.0)
    p = p * r + jnp.float32(1.0 / 120.0)
    p = p * r + jnp.float32(1.0 / 24.0)
    p = p * r + jnp.float32(1.0 / 6.0)
    p = p * r + jnp.float32(0.5)
    p = p * r + jnp.float32(1.0)
    p = p * r + jnp.float32(1.0)
    ni = n.astype(jnp.int32)
    sc = plsc.bitcast((ni + 127) << 23, jnp.float32)
    return p * sc


def _stage_b_body(src_hbm, dst_hbm, qs_hbm, t2_hbm, v_hbm, z128_hbm,
                  opart_hbm,
                  src_v, dst_v, qs_v, t2_v, vv_v, pay_v, abuf, osh):
    cid = lax.axis_index("c")
    sid = lax.axis_index("s")
    wid = cid * NS + sid

    rows0 = sid * RPS
    pltpu.sync_copy(z128_hbm.at[pl.ds(rows0, RPS)], osh.at[pl.ds(rows0, RPS)])
    plsc.subcore_barrier()

    iota = lax.iota(jnp.int32, 16)

    def chunk(i, carry):
        off = wid * EPW + i * CH
        pltpu.sync_copy(src_hbm.at[pl.ds(off, CH)], src_v)
        pltpu.sync_copy(dst_hbm.at[pl.ds(off, CH)], dst_v)
        pltpu.sync_copy(qs_hbm.at[src_v], qs_v)
        pltpu.sync_copy(t2_hbm.at[dst_v], t2_v)
        pltpu.sync_copy(v_hbm.at[dst_v], vv_v)
        for g in range(CH // 16):
            rows = iota + g * 16
            qsv = [plsc.load_gather(qs_v, [rows, jnp.full((16,), hh, jnp.int32)])
                   for hh in range(H)]
            ap = [qsv[hh] * plsc.load_gather(
                      t2_v, [rows, jnp.full((16,), hh, jnp.int32)])
                  for hh in range(H)]
            an = [qsv[hh] * plsc.load_gather(
                      t2_v, [rows, jnp.full((16,), H + hh, jnp.int32)])
                  for hh in range(H)]
            mp = ap[0]
            mn = an[0]
            for hh in range(1, H):
                mp = jnp.maximum(mp, ap[hh])
                mn = jnp.maximum(mn, an[hh])
            ep = [_exp16(a - mp) for a in ap]
            en = [_exp16(a - mn) for a in an]
            sp = ep[0]
            sn = en[0]
            for hh in range(1, H):
                sp = sp + ep[hh]
                sn = sn + en[hh]
            rp = 1.0 / sp
            rn = 1.0 / sn
            for hh in range(H):
                abuf[hh, pl.ds(16, 16)] = ep[hh] * rp - en[hh] * rn
            for e in range(16):
                r = g * 16 + e
                for hh in range(H):
                    b = plsc.load_gather(
                        abuf, [jnp.full((16,), hh, jnp.int32),
                               jnp.full((16,), 16 + e, jnp.int32)])
                    pay_v[r, pl.ds(hh * D, D)] = vv_v[r, pl.ds(hh * D, D)] * b
        pltpu.sync_copy(pay_v, osh.at[src_v], add=True)
        return carry

    lax.fori_loop(0, NCHUNK, chunk, 0)
    plsc.subcore_barrier()
    pltpu.sync_copy(osh.at[pl.ds(rows0, RPS)],
                    opart_hbm.at[cid, pl.ds(rows0, RPS)])


def _k4(src, dst, qs, t2, v, z128):
    f = pl.kernel(
        _stage_b_body,
        out_type=jax.ShapeDtypeStruct((NC, NP, IND), jnp.float32),
        mesh=_mesh(),
        compiler_params=pltpu.CompilerParams(needs_layout_passes=False, use_tc_tiling_on_sc=False),
        scratch_types=[
            pltpu.VMEM((CH,), jnp.int32),
            pltpu.VMEM((CH,), jnp.int32),
            pltpu.VMEM((CH, 2 * H), jnp.float32),
            pltpu.VMEM((CH, 2 * H), jnp.float32),
            pltpu.VMEM((CH, IND), jnp.float32),
            pltpu.VMEM((CH, IND), jnp.float32),
            pltpu.VMEM((H, 32), jnp.float32),
            pltpu.VMEM_SHARED((NP, IND), jnp.float32),
        ],
    )
    return f(src, dst, qs, t2, v, z128)


def _add_body(p_ref, o_ref):
    o_ref[...] = p_ref[0] + p_ref[1]


def _k5(opart):
    blk = 2000
    g = N // blk
    return pl.pallas_call(
        _add_body,
        grid=(g,),
        in_specs=[pl.BlockSpec((NC, blk, IND), lambda i: (0, i, 0))],
        out_specs=pl.BlockSpec((blk, IND), lambda i: (i, 0)),
        out_shape=jax.ShapeDtypeStruct((N, IND), jnp.float32),
    )(opart)


def kernel(x, edge_index, edge_sign, Wq, Wk, Wv, RF, RF_q):
    bdq = jnp.kron(jnp.eye(H, dtype=jnp.float32), RF_q)
    bdk = jnp.kron(jnp.eye(H, dtype=jnp.float32), RF)

    v, qs, phi64 = _k1(x, Wv, Wq, Wk, bdq, bdk)
    phi_s = phi64[:N // H].reshape(N, H)

    z32 = jnp.zeros((NP, 32), jnp.float32)
    z128 = jnp.zeros((NP, IND), jnp.float32)

    src = edge_index[0]
    dst = edge_index[1]
    tpart = _k2(src, dst, edge_sign, phi_s, z32)
    t2 = _k3(tpart)
    opart = _k4(src, dst, qs, t2, v, z128)
    return _k5(opart)


# speedup vs baseline: 49.9798x; 1.9622x over previous
import functools

import jax
import jax.numpy as jnp
from jax import lax
from jax.experimental import pallas as pl
from jax.experimental.pallas import tpu as pltpu
from jax.experimental.pallas import tpu_sc as plsc

N = 10000
E = 320000
H = 8
D = 16
M = 8
IND = 128

NC = 2
NS = 16
NW = NC * NS
EPW = E // NW
NP = 10240
CH = 80
NCHUNK = EPW // CH
RPS = NP // NS

@functools.cache
def _mesh():
    return plsc.VectorSubcoreMesh(core_axis_name="c", subcore_axis_name="s",
                                  num_cores=NC, num_subcores=NS)


def _pre_body(x_ref, wv_ref, wq_ref, wk_ref, bdq_ref, bdk_ref,
              v_ref, qs_ref, phi_ref):
    xb = x_ref[...]
    v_ref[...] = jnp.dot(xb, wv_ref[...], preferred_element_type=jnp.float32)
    qb = jnp.dot(xb, wq_ref[...], preferred_element_type=jnp.float32)
    qr = jnp.dot(qb, bdq_ref[...], preferred_element_type=jnp.float32)
    cs = jnp.cos(qr) + jnp.sin(qr)
    r8 = lax.broadcasted_iota(jnp.int32, (H * M, 2 * H), 0) // M
    c8 = lax.broadcasted_iota(jnp.int32, (H * M, 2 * H), 1)
    sel = (r8 == c8).astype(jnp.float32)
    qs_ref[...] = jnp.dot(cs, sel, preferred_element_type=jnp.float32)
    kb = jnp.dot(xb, wk_ref[...], preferred_element_type=jnp.float32)
    kr = jnp.dot(kb, bdk_ref[...], preferred_element_type=jnp.float32)
    phi_ref[...] = jnp.cos(kr) + jnp.sin(kr)


def _k1(x, wv, wq, wk, bdq, bdk):
    blk = 2000
    g = N // blk
    return pl.pallas_call(
        _pre_body,
        grid=(g,),
        in_specs=[
            pl.BlockSpec((blk, IND), lambda i: (i, 0)),
            pl.BlockSpec((IND, IND), lambda i: (0, 0)),
            pl.BlockSpec((IND, IND), lambda i: (0, 0)),
            pl.BlockSpec((IND, IND), lambda i: (0, 0)),
            pl.BlockSpec((IND, H * M), lambda i: (0, 0)),
            pl.BlockSpec((IND, H * M), lambda i: (0, 0)),
        ],
        out_specs=[
            pl.BlockSpec((blk, IND), lambda i: (i, 0)),
            pl.BlockSpec((blk, 2 * H), lambda i: (i, 0)),
            pl.BlockSpec((blk, H * M), lambda i: (i, 0)),
        ],
        out_shape=[
            jax.ShapeDtypeStruct((N, IND), jnp.float32),
            jax.ShapeDtypeStruct((N, 2 * H), jnp.float32),
            jax.ShapeDtypeStruct((N, H * M), jnp.float32),
        ],
    )(x, wv, wq, wk, bdq, bdk)


def _stage_a_body(src_hbm, dst_hbm, es_hbm, phi_hbm, z32_hbm, tpart_hbm,
                  s0, s1, d0, d1, g0, g1, x0, x1, f0, f1, p0, p1, tsh,
                  si0, si1, sg0, sg1, ss0, ss1):
    SRC = [s0, s1]
    DST = [d0, d1]
    SGN = [g0, g1]
    SSI = [x0, x1]
    PHR = [f0, f1]
    PAY = [p0, p1]
    SI = [si0, si1]
    SG = [sg0, sg1]
    SS = [ss0, ss1]

    cid = lax.axis_index("c")
    sid = lax.axis_index("s")
    wid = cid * NS + sid

    rows0 = sid * RPS
    pltpu.sync_copy(z32_hbm.at[pl.ds(rows0, RPS)], tsh.at[pl.ds(rows0, RPS)])

    iota = lax.iota(jnp.int32, 16)
    zero16 = jnp.zeros((16,), jnp.float32)
    for b in range(2):
        for g in range(CH // 16):
            rows = iota + g * 16
            for cz in range(18, 32):
                plsc.store_scatter(PAY[b],
                                   [rows, jnp.full((16,), cz, jnp.int32)],
                                   zero16)
    plsc.subcore_barrier()

    def off_of(c):
        return wid * EPW + jnp.minimum(c, NCHUNK - 1) * CH

    def issue_idx(c, b):
        off = off_of(c)
        pltpu.async_copy(src_hbm.at[pl.ds(off, CH)], SRC[b], SI[b])
        pltpu.async_copy(dst_hbm.at[pl.ds(off, CH)], DST[b], SI[b])
        pltpu.async_copy(es_hbm.at[pl.ds(off, CH)], SGN[b], SI[b])

    def wait_idx(b):
        pltpu.make_async_copy(src_hbm.at[pl.ds(0, CH)], SRC[b], SI[b]).wait()
        pltpu.make_async_copy(dst_hbm.at[pl.ds(0, CH)], DST[b], SI[b]).wait()
        pltpu.make_async_copy(es_hbm.at[pl.ds(0, CH)], SGN[b], SI[b]).wait()

    def issue_gat(b):
        pltpu.async_copy(phi_hbm.at[SRC[b]], PHR[b], SG[b])

    def wait_gat(b):
        pltpu.make_async_copy(phi_hbm.at[SRC[b]], PHR[b], SG[b]).wait()

    def issue_scat(b):
        pltpu.async_copy(PAY[b], tsh.at[SSI[b]], SS[b], add=True)

    def wait_scat(b):
        pltpu.make_async_copy(PAY[b], tsh.at[SSI[b]], SS[b]).wait()

    def compute(b):
        phr = PHR[b]
        pay = PAY[b]
        sgn = SGN[b]

        def group(g, carry):
            rows = iota + g * 16
            sg = plsc.load_gather(sgn, [rows])
            pf = jnp.where(sg > 0, 1.0, 0.0)
            nf = 1.0 - pf
            for hh in range(H):
                ph = plsc.load_gather(phr,
                                      [rows, jnp.full((16,), hh, jnp.int32)])
                plsc.store_scatter(pay,
                                   [rows, jnp.full((16,), hh, jnp.int32)],
                                   ph * pf)
                plsc.store_scatter(pay,
                                   [rows, jnp.full((16,), H + hh, jnp.int32)],
                                   ph * nf)
            plsc.store_scatter(pay, [rows, jnp.full((16,), 16, jnp.int32)], pf)
            plsc.store_scatter(pay, [rows, jnp.full((16,), 17, jnp.int32)], nf)
            return carry

        lax.fori_loop(0, CH // 16, group, 0)
        for z in range(CH // 16):
            SSI[b][pl.ds(z * 16, 16)] = DST[b][pl.ds(z * 16, 16)]

    def body(c, b):
        nb = 1 - b
        wait_idx(nb)
        issue_gat(nb)
        wait_gat(b)

        @pl.when(c >= 2)
        def _():
            wait_scat(b)

        compute(b)
        issue_scat(b)

        @pl.when(c + 2 <= NCHUNK - 1)
        def _():
            issue_idx(c + 2, b)

    issue_idx(0, 0)
    wait_idx(0)
    issue_gat(0)
    issue_idx(1, 1)

    def pair(j, carry):
        body(2 * j, 0)
        body(2 * j + 1, 1)
        return carry

    lax.fori_loop(0, (NCHUNK - 1) // 2, pair, 0)
    wait_gat(0)
    wait_scat(0)
    compute(0)
    issue_scat(0)
    wait_scat(1)
    wait_scat(0)

    plsc.subcore_barrier()
    pltpu.sync_copy(tsh.at[pl.ds(rows0, RPS)],
                    tpart_hbm.at[cid, pl.ds(rows0, RPS)])


def _k2(src, dst, edge_sign, phi_s, z32):
    f = pl.kernel(
        _stage_a_body,
        out_type=jax.ShapeDtypeStruct((NC, NP, 32), jnp.float32),
        mesh=_mesh(),
        compiler_params=pltpu.CompilerParams(needs_layout_passes=False, use_tc_tiling_on_sc=False),
        scratch_types=[
            pltpu.VMEM((CH,), jnp.int32),
            pltpu.VMEM((CH,), jnp.int32),
            pltpu.VMEM((CH,), jnp.int32),
            pltpu.VMEM((CH,), jnp.int32),
            pltpu.VMEM((CH,), jnp.int32),
            pltpu.VMEM((CH,), jnp.int32),
            pltpu.VMEM((CH,), jnp.int32),
            pltpu.VMEM((CH,), jnp.int32),
            pltpu.VMEM((CH, H), jnp.float32),
            pltpu.VMEM((CH, H), jnp.float32),
            pltpu.VMEM((CH, 32), jnp.float32),
            pltpu.VMEM((CH, 32), jnp.float32),
            pltpu.VMEM_SHARED((NP, 32), jnp.float32),
            pltpu.SemaphoreType.DMA,
            pltpu.SemaphoreType.DMA,
            pltpu.SemaphoreType.DMA,
            pltpu.SemaphoreType.DMA,
            pltpu.SemaphoreType.DMA,
            pltpu.SemaphoreType.DMA,
        ],
    )
    return f(src, dst, edge_sign, phi_s, z32)


def _t2_body(tp_ref, t2_ref):
    t = tp_ref[0] + tp_ref[1]
    cp = t[:, 0:H]
    cn = t[:, H:2 * H]
    dp = jnp.maximum(t[:, 16:17], 1.0)
    dn = jnp.maximum(t[:, 17:18], 1.0)
    t2_ref[...] = jnp.concatenate([cp / dp, cn / dn], axis=1)


def _k3(tpart):
    blk = 2000
    g = N // blk
    return pl.pallas_call(
        _t2_body,
        grid=(g,),
        in_specs=[pl.BlockSpec((NC, blk, 32), lambda i: (0, i, 0))],
        out_specs=pl.BlockSpec((blk, 2 * H), lambda i: (i, 0)),
        out_shape=jax.ShapeDtypeStruct((N, 2 * H), jnp.float32),
    )(tpart)


_LOG2E = 1.4426950408889634
_LN2_HI = 0.6931471824645996
_LN2_LO = -1.904654323148236e-09
_RND = 12582912.0


def _exp16(x):
    t = x * _LOG2E + _RND
    n = t - _RND
    r = x - n * _LN2_HI
    r = r - n * _LN2_LO
    p = jnp.float32(1.0 / 720.0)
    p = p * r + jnp.float32(1.0 / 120.0)
    p = p * r + jnp.float32(1.0 / 24.0)
    p = p * r + jnp.float32(1.0 / 6.0)
    p = p * r + jnp.float32(0.5)
    p = p * r + jnp.float32(1.0)
    p = p * r + jnp.float32(1.0)
    ni = n.astype(jnp.int32)
    sc = plsc.bitcast((ni + 127) << 23, jnp.float32)
    return p * sc


def _stage_b_body(src_hbm, dst_hbm, qs_hbm, t2_hbm, v_hbm, z128_hbm,
                  opart_hbm,
                  s0, s1, d0, d1, x0, x1, q0, q1, t0, t1, v0, v1, p0, p1,
                  abuf, osh, si0, si1, sg0, sg1, ss0, ss1):
    SRC = [s0, s1]
    DST = [d0, d1]
    SSI = [x0, x1]
    QS = [q0, q1]
    T2 = [t0, t1]
    VV = [v0, v1]
    PAY = [p0, p1]
    SI = [si0, si1]
    SG = [sg0, sg1]
    SS = [ss0, ss1]

    cid = lax.axis_index("c")
    sid = lax.axis_index("s")
    wid = cid * NS + sid

    rows0 = sid * RPS
    pltpu.sync_copy(z128_hbm.at[pl.ds(rows0, RPS)], osh.at[pl.ds(rows0, RPS)])
    plsc.subcore_barrier()

    iota = lax.iota(jnp.int32, 16)

    def off_of(c):
        return wid * EPW + jnp.minimum(c, NCHUNK - 1) * CH

    def issue_idx(c, b):
        off = off_of(c)
        pltpu.async_copy(src_hbm.at[pl.ds(off, CH)], SRC[b], SI[b])
        pltpu.async_copy(dst_hbm.at[pl.ds(off, CH)], DST[b], SI[b])

    def wait_idx(b):
        pltpu.make_async_copy(src_hbm.at[pl.ds(0, CH)], SRC[b], SI[b]).wait()
        pltpu.make_async_copy(dst_hbm.at[pl.ds(0, CH)], DST[b], SI[b]).wait()

    def issue_gat(b):
        pltpu.async_copy(qs_hbm.at[SRC[b]], QS[b], SG[b])
        pltpu.async_copy(t2_hbm.at[DST[b]], T2[b], SG[b])
        pltpu.async_copy(v_hbm.at[DST[b]], VV[b], SG[b])

    def wait_gat(b):
        pltpu.make_async_copy(qs_hbm.at[SRC[b]], QS[b], SG[b]).wait()
        pltpu.make_async_copy(t2_hbm.at[DST[b]], T2[b], SG[b]).wait()
        pltpu.make_async_copy(v_hbm.at[DST[b]], VV[b], SG[b]).wait()

    def issue_scat(b):
        pltpu.async_copy(PAY[b], osh.at[SSI[b]], SS[b], add=True)

    def wait_scat(b):
        pltpu.make_async_copy(PAY[b], osh.at[SSI[b]], SS[b]).wait()

    def compute(b):
        t2b = T2[b]
        vvb = VV[b]
        qsb = QS[b]
        pay = PAY[b]

        def group(g, carry):
            rows = iota + g * 16
            qsv = [plsc.load_gather(qsb, [rows, jnp.full((16,), hh, jnp.int32)])
                   for hh in range(H)]
            ap = [qsv[hh] * plsc.load_gather(
                      t2b, [rows, jnp.full((16,), hh, jnp.int32)])
                  for hh in range(H)]
            an = [qsv[hh] * plsc.load_gather(
                      t2b, [rows, jnp.full((16,), H + hh, jnp.int32)])
                  for hh in range(H)]
            mp = ap[0]
            mn = an[0]
            for hh in range(1, H):
                mp = jnp.maximum(mp, ap[hh])
                mn = jnp.maximum(mn, an[hh])
            ep = [_exp16(a - mp) for a in ap]
            en = [_exp16(a - mn) for a in an]
            sp = ep[0]
            sn = en[0]
            for hh in range(1, H):
                sp = sp + ep[hh]
                sn = sn + en[hh]
            rp = 1.0 / sp
            rn = 1.0 / sn
            for hh in range(H):
                abuf[hh, pl.ds(16, 16)] = ep[hh] * rp - en[hh] * rn
            for e in range(16):
                rfull = jnp.zeros((16,), jnp.int32) + (g * 16 + e)
                ecol = jnp.full((16,), 16 + e, jnp.int32)
                for hh in range(H):
                    bv = plsc.load_gather(
                        abuf, [jnp.full((16,), hh, jnp.int32), ecol])
                    seg = plsc.load_gather(vvb, [rfull, hh * D + iota])
                    plsc.store_scatter(pay, [rfull, hh * D + iota], seg * bv)
            return carry

        lax.fori_loop(0, CH // 16, group, 0)
        for z in range(CH // 16):
            SSI[b][pl.ds(z * 16, 16)] = SRC[b][pl.ds(z * 16, 16)]

    def body(c, b):
        nb = 1 - b
        wait_idx(nb)
        issue_gat(nb)
        wait_gat(b)

        @pl.when(c >= 2)
        def _():
            wait_scat(b)

        compute(b)
        issue_scat(b)

        @pl.when(c + 2 <= NCHUNK - 1)
        def _():
            issue_idx(c + 2, b)

    issue_idx(0, 0)
    wait_idx(0)
    issue_gat(0)
    issue_idx(1, 1)

    def pair(j, carry):
        body(2 * j, 0)
        body(2 * j + 1, 1)
        return carry

    lax.fori_loop(0, (NCHUNK - 1) // 2, pair, 0)
    wait_gat(0)
    wait_scat(0)
    compute(0)
    issue_scat(0)
    wait_scat(1)
    wait_scat(0)

    plsc.subcore_barrier()
    pltpu.sync_copy(osh.at[pl.ds(rows0, RPS)],
                    opart_hbm.at[cid, pl.ds(rows0, RPS)])


def _k4(src, dst, qs, t2, v, z128):
    f = pl.kernel(
        _stage_b_body,
        out_type=jax.ShapeDtypeStruct((NC, NP, IND), jnp.float32),
        mesh=_mesh(),
        compiler_params=pltpu.CompilerParams(needs_layout_passes=False, use_tc_tiling_on_sc=False),
        scratch_types=[
            pltpu.VMEM((CH,), jnp.int32),
            pltpu.VMEM((CH,), jnp.int32),
            pltpu.VMEM((CH,), jnp.int32),
            pltpu.VMEM((CH,), jnp.int32),
            pltpu.VMEM((CH,), jnp.int32),
            pltpu.VMEM((CH,), jnp.int32),
            pltpu.VMEM((CH, 2 * H), jnp.float32),
            pltpu.VMEM((CH, 2 * H), jnp.float32),
            pltpu.VMEM((CH, 2 * H), jnp.float32),
            pltpu.VMEM((CH, 2 * H), jnp.float32),
            pltpu.VMEM((CH, IND), jnp.float32),
            pltpu.VMEM((CH, IND), jnp.float32),
            pltpu.VMEM((CH, IND), jnp.float32),
            pltpu.VMEM((CH, IND), jnp.float32),
            pltpu.VMEM((H, 32), jnp.float32),
            pltpu.VMEM_SHARED((NP, IND), jnp.float32),
            pltpu.SemaphoreType.DMA,
            pltpu.SemaphoreType.DMA,
            pltpu.SemaphoreType.DMA,
            pltpu.SemaphoreType.DMA,
            pltpu.SemaphoreType.DMA,
            pltpu.SemaphoreType.DMA,
        ],
    )
    return f(src, dst, qs, t2, v, z128)


def _add_body(p_ref, o_ref):
    o_ref[...] = p_ref[0] + p_ref[1]


def _k5(opart):
    blk = 2000
    g = N // blk
    return pl.pallas_call(
        _add_body,
        grid=(g,),
        in_specs=[pl.BlockSpec((NC, blk, IND), lambda i: (0, i, 0))],
        out_specs=pl.BlockSpec((blk, IND), lambda i: (i, 0)),
        out_shape=jax.ShapeDtypeStruct((N, IND), jnp.float32),
    )(opart)


def kernel(x, edge_index, edge_sign, Wq, Wk, Wv, RF, RF_q):
    bdq = jnp.kron(jnp.eye(H, dtype=jnp.float32), RF_q)
    bdk = jnp.kron(jnp.eye(H, dtype=jnp.float32), RF)

    v, qs, phi64 = _k1(x, Wv, Wq, Wk, bdq, bdk)
    phi_s = phi64[:N // H].reshape(N, H)

    z32 = jnp.zeros((NP, 32), jnp.float32)
    z128 = jnp.zeros((NP, IND), jnp.float32)

    src = edge_index[0]
    dst = edge_index[1]
    tpart = _k2(src, dst, edge_sign, phi_s, z32)
    t2 = _k3(tpart)
    opart = _k4(src, dst, qs, t2, v, z128)
    return _k5(opart)
